# baseline (speedup 1.0000x reference)
import functools

import jax
import jax.numpy as jnp
from jax import lax
from jax.experimental import pallas as pl
from jax.experimental.pallas import tpu as pltpu
from jax.experimental.pallas import tpu_sc as plsc

T = 4096
D = 1024
E = 8
H = 2816
K = 2
R = 512
P = T * K + E * R // 1
NT = P // R
TE_PAD = 128

NW = 32
CH_D = 32
NCH_D = T // (NW * CH_D)
CH_C = 16
NCH_C = T // (NW * CH_C)


def _router_body(x_ref, wr_ref, d0_ref, d1_ref, w0_ref, w1_ref, te_ref):
    x = x_ref[...]
    wr = wr_ref[...]
    logits = lax.dot_general(wr, x, (((1,), (1,)), ((), ())),
                             precision=lax.Precision.DEFAULT,
                             preferred_element_type=jnp.float32)
    i0 = lax.broadcasted_iota(jnp.int32, (E, T), 0)

    m0 = jnp.max(logits, axis=0, keepdims=True)
    e0 = jnp.min(jnp.where(logits == m0, i0, E), axis=0, keepdims=True)
    masked = jnp.where(i0 == e0, -jnp.inf, logits)
    m1 = jnp.max(masked, axis=0, keepdims=True)
    e1 = jnp.min(jnp.where(masked == m1, i0, E), axis=0, keepdims=True)

    s = jnp.exp(m1 - m0)
    w0 = 1.0 / (1.0 + s)
    w1 = s * w0

    oh0 = (i0 == e0).astype(jnp.float32)
    oh1 = (i0 == e1).astype(jnp.float32)
    c = oh0 + oh1

    c2 = c.reshape(E * 32, 128)
    ju = lax.broadcasted_iota(jnp.int32, (128, 128), 0)
    iu = lax.broadcasted_iota(jnp.int32, (128, 128), 1)
    u128 = (ju < iu).astype(jnp.float32)
    s_in = lax.dot_general(c2, u128, (((1,), (0,)), ((), ())),
                           preferred_element_type=jnp.float32)
    tb = lax.broadcasted_iota(jnp.int32, (32, T), 0)
    tt = lax.shift_right_logical(lax.broadcasted_iota(jnp.int32, (32, T), 1), 7)
    expm = (tb == tt).astype(jnp.float32)
    tot2 = lax.dot_general(c, expm, (((1,), (1,)), ((), ())),
                           preferred_element_type=jnp.float32)
    jb = lax.broadcasted_iota(jnp.int32, (32, 32), 0)
    ib = lax.broadcasted_iota(jnp.int32, (32, 32), 1)
    u32 = (jb < ib).astype(jnp.float32)
    off = lax.dot_general(tot2, u32, (((1,), (0,)), ((), ())),
                          preferred_element_type=jnp.float32)
    off_exp = lax.dot_general(off, expm, (((1,), (0,)), ((), ())),
                              precision=lax.Precision.HIGHEST,
                              preferred_element_type=jnp.float32)
    s_full = s_in.reshape(E, T) + off_exp

    counts = jnp.sum(c, axis=1, keepdims=True)
    pc = jnp.floor((counts + (R - 1)) / R) * R
    je = lax.broadcasted_iota(jnp.int32, (E, E), 0)
    ie = lax.broadcasted_iota(jnp.int32, (E, E), 1)
    l8 = (ie < je).astype(jnp.float32)
    starts = lax.dot_general(l8, pc, (((1,), (0,)), ((), ())),
                             precision=lax.Precision.HIGHEST,
                             preferred_element_type=jnp.float32)
    ends = starts + pc

    pos = s_full + starts
    d0 = jnp.sum(oh0 * pos, axis=0, keepdims=True)
    d1 = jnp.sum(oh1 * pos, axis=0, keepdims=True)

    jt = lax.broadcasted_iota(jnp.int32, (E, TE_PAD), 1).astype(jnp.float32) * R
    te = jnp.sum((ends <= jt).astype(jnp.int32), axis=0, keepdims=True)
    te = jnp.minimum(te, E - 1)

    d0_ref[...] = d0.astype(jnp.int32)
    d1_ref[...] = d1.astype(jnp.int32)
    w0_ref[...] = w0
    w1_ref[...] = w1
    te_ref[...] = te


def _router_call(x_flat, wr):
    outs = pl.pallas_call(
        _router_body,
        out_shape=(
            jax.ShapeDtypeStruct((1, T), jnp.int32),
            jax.ShapeDtypeStruct((1, T), jnp.int32),
            jax.ShapeDtypeStruct((1, T), jnp.float32),
            jax.ShapeDtypeStruct((1, T), jnp.float32),
            jax.ShapeDtypeStruct((1, TE_PAD), jnp.int32),
        ),
    )(x_flat, wr)
    return outs


@functools.cache
def _dispatch_sc_kernel():
    @functools.partial(
        pl.kernel,
        mesh=plsc.VectorSubcoreMesh(core_axis_name="c", subcore_axis_name="s"),
        out_type=jax.ShapeDtypeStruct((P, D), jnp.float32),
        scratch_types=[
            pltpu.VMEM((CH_D, D), jnp.float32),
            pltpu.VMEM((CH_D,), jnp.int32),
            pltpu.VMEM((CH_D,), jnp.int32),
            pltpu.SemaphoreType.DMA,
        ],
    )
    def _dispatch_sc(x_hbm, d0_hbm, d1_hbm, xpad_hbm,
                     xbuf, idx0, idx1, sem):
        wid = lax.axis_index("s") * 2 + lax.axis_index("c")
        for c in range(NCH_D):
            base = wid * (NCH_D * CH_D) + c * CH_D
            pltpu.sync_copy(x_hbm.at[pl.ds(base, CH_D)], xbuf)
            pltpu.sync_copy(d0_hbm.at[wid, c], idx0)
            pltpu.sync_copy(d1_hbm.at[wid, c], idx1)
            a = pltpu.async_copy(xbuf, xpad_hbm.at[idx0], sem)
            b = pltpu.async_copy(xbuf, xpad_hbm.at[idx1], sem)
            a.wait()
            b.wait()

    return _dispatch_sc


def _ffn_body(te_ref, x_ref, w1_ref, w3_ref, w2_ref, o_ref):
    x = x_ref[...].astype(jnp.bfloat16)
    w1 = w1_ref[0]
    w3 = w3_ref[0]
    w2 = w2_ref[0]
    h1 = lax.dot_general(x, w1, (((1,), (1,)), ((), ())),
                         preferred_element_type=jnp.float32)
    h3 = lax.dot_general(x, w3, (((1,), (1,)), ((), ())),
                         preferred_element_type=jnp.float32)
    h = (h1 * jax.nn.sigmoid(h1) * h3).astype(jnp.bfloat16)
    y = lax.dot_general(h, w2, (((1,), (1,)), ((), ())),
                        preferred_element_type=jnp.float32)
    o_ref[...] = y


def _ffn_call(x_pad, te, w1, w2, w3):
    grid_spec = pltpu.PrefetchScalarGridSpec(
        num_scalar_prefetch=1,
        grid=(NT,),
        in_specs=[
            pl.BlockSpec((R, D), lambda i, te: (i, 0)),
            pl.BlockSpec((1, H, D), lambda i, te: (te[i], 0, 0)),
            pl.BlockSpec((1, H, D), lambda i, te: (te[i], 0, 0)),
            pl.BlockSpec((1, D, H), lambda i, te: (te[i], 0, 0)),
        ],
        out_specs=pl.BlockSpec((R, D), lambda i, te: (i, 0)),
    )
    return pl.pallas_call(
        _ffn_body,
        grid_spec=grid_spec,
        out_shape=jax.ShapeDtypeStruct((P, D), jnp.float32),
        compiler_params=pltpu.CompilerParams(
            dimension_semantics=("arbitrary",),
        ),
    )(te, x_pad, w1, w3, w2)


@functools.cache
def _combine_sc_kernel():
    @functools.partial(
        pl.kernel,
        mesh=plsc.VectorSubcoreMesh(core_axis_name="c", subcore_axis_name="s"),
        out_type=jax.ShapeDtypeStruct((T, D), jnp.float32),
        scratch_types=[
            pltpu.VMEM((CH_C,), jnp.int32),
            pltpu.VMEM((CH_C,), jnp.int32),
            pltpu.VMEM((CH_C,), jnp.float32),
            pltpu.VMEM((CH_C,), jnp.float32),
            pltpu.VMEM((CH_C, D), jnp.float32),
            pltpu.VMEM((CH_C, D), jnp.float32),
            pltpu.SemaphoreType.DMA,
        ],
    )
    def _combine_sc(yp_hbm, d0_hbm, d1_hbm, w0_hbm, w1_hbm, out_hbm,
                    idx0, idx1, w0v, w1v, b0, b1, sem):
        wid = lax.axis_index("s") * 2 + lax.axis_index("c")
        for c in range(NCH_C):
            base = wid * (NCH_C * CH_C) + c * CH_C
            pltpu.sync_copy(d0_hbm.at[wid, c], idx0)
            pltpu.sync_copy(d1_hbm.at[wid, c], idx1)
            pltpu.sync_copy(w0_hbm.at[wid, c], w0v)
            pltpu.sync_copy(w1_hbm.at[wid, c], w1v)
            a = pltpu.async_copy(yp_hbm.at[idx0], b0, sem)
            b = pltpu.async_copy(yp_hbm.at[idx1], b1, sem)
            a.wait()
            b.wait()

            w0vec = w0v[...]
            w1vec = w1v[...]
            ws0 = [w0vec[r] for r in range(CH_C)]
            ws1 = [w1vec[r] for r in range(CH_C)]

            def body(cc, _):
                sl = pl.ds(cc * 16, 16)
                for r in range(CH_C):
                    b0[r, sl] = ws0[r] * b0[r, sl] + ws1[r] * b1[r, sl]
                return 0

            lax.fori_loop(0, D // 16, body, 0)
            pltpu.sync_copy(b0, out_hbm.at[pl.ds(base, CH_C)])

    return _combine_sc


def kernel(x, Wr, W1, W2, W3):
    b, s, d = x.shape
    x_flat = x.reshape(T, D)
    d0, d1, w0, w1, te = _router_call(x_flat, Wr)

    d0d = d0.reshape(NW, NCH_D, CH_D)
    d1d = d1.reshape(NW, NCH_D, CH_D)
    x_pad = _dispatch_sc_kernel()(x_flat, d0d, d1d)

    y = _ffn_call(x_pad, te.reshape(TE_PAD),
                  W1.astype(jnp.bfloat16),
                  W2.astype(jnp.bfloat16),
                  W3.astype(jnp.bfloat16))

    d0c = d0.reshape(NW, NCH_C, CH_C)
    d1c = d1.reshape(NW, NCH_C, CH_C)
    w0c = w0.reshape(NW, NCH_C, CH_C)
    w1c = w1.reshape(NW, NCH_C, CH_C)
    out = _combine_sc_kernel()(y, d0c, d1c, w0c, w1c)
    return out.reshape(b, s, d)


# speedup vs baseline: 1.0911x; 1.0911x over previous
import functools

import jax
import jax.numpy as jnp
from jax import lax
from jax.experimental import pallas as pl
from jax.experimental.pallas import tpu as pltpu
from jax.experimental.pallas import tpu_sc as plsc

T = 4096
D = 1024
E = 8
H = 2816
K = 2
R = 256
P = T * K + E * R // 1
NT = P // R
TE_PAD = 128

NW = 32
CH_D = 32
NCH_D = T // (NW * CH_D)
CH_C = 16
NCH_C = T // (NW * CH_C)


def _router_body(x_ref, wr_ref, d0_ref, d1_ref, w0_ref, w1_ref, te_ref):
    x = x_ref[...]
    wr = wr_ref[...]
    logits = lax.dot_general(wr, x, (((1,), (1,)), ((), ())),
                             precision=lax.Precision.DEFAULT,
                             preferred_element_type=jnp.float32)
    i0 = lax.broadcasted_iota(jnp.int32, (E, T), 0)

    m0 = jnp.max(logits, axis=0, keepdims=True)
    e0 = jnp.min(jnp.where(logits == m0, i0, E), axis=0, keepdims=True)
    masked = jnp.where(i0 == e0, -jnp.inf, logits)
    m1 = jnp.max(masked, axis=0, keepdims=True)
    e1 = jnp.min(jnp.where(masked == m1, i0, E), axis=0, keepdims=True)

    s = jnp.exp(m1 - m0)
    w0 = 1.0 / (1.0 + s)
    w1 = s * w0

    oh0 = (i0 == e0).astype(jnp.float32)
    oh1 = (i0 == e1).astype(jnp.float32)
    c = oh0 + oh1

    c2 = c.reshape(E * 32, 128)
    ju = lax.broadcasted_iota(jnp.int32, (128, 128), 0)
    iu = lax.broadcasted_iota(jnp.int32, (128, 128), 1)
    u128 = (ju < iu).astype(jnp.float32)
    s_in = lax.dot_general(c2, u128, (((1,), (0,)), ((), ())),
                           preferred_element_type=jnp.float32)
    tb = lax.broadcasted_iota(jnp.int32, (32, T), 0)
    tt = lax.shift_right_logical(lax.broadcasted_iota(jnp.int32, (32, T), 1), 7)
    expm = (tb == tt).astype(jnp.float32)
    tot2 = lax.dot_general(c, expm, (((1,), (1,)), ((), ())),
                           preferred_element_type=jnp.float32)
    jb = lax.broadcasted_iota(jnp.int32, (32, 32), 0)
    ib = lax.broadcasted_iota(jnp.int32, (32, 32), 1)
    u32 = (jb < ib).astype(jnp.float32)
    off = lax.dot_general(tot2, u32, (((1,), (0,)), ((), ())),
                          preferred_element_type=jnp.float32)
    off_exp = lax.dot_general(off, expm, (((1,), (0,)), ((), ())),
                              precision=lax.Precision.HIGHEST,
                              preferred_element_type=jnp.float32)
    s_full = s_in.reshape(E, T) + off_exp

    counts = jnp.sum(c, axis=1, keepdims=True)
    pc = jnp.floor((counts + (R - 1)) / R) * R
    je = lax.broadcasted_iota(jnp.int32, (E, E), 0)
    ie = lax.broadcasted_iota(jnp.int32, (E, E), 1)
    l8 = (ie < je).astype(jnp.float32)
    starts = lax.dot_general(l8, pc, (((1,), (0,)), ((), ())),
                             precision=lax.Precision.HIGHEST,
                             preferred_element_type=jnp.float32)
    ends = starts + pc

    pos = s_full + starts
    d0 = jnp.sum(oh0 * pos, axis=0, keepdims=True)
    d1 = jnp.sum(oh1 * pos, axis=0, keepdims=True)

    jt = lax.broadcasted_iota(jnp.int32, (E, TE_PAD), 1).astype(jnp.float32) * R
    te = jnp.sum((ends <= jt).astype(jnp.int32), axis=0, keepdims=True)
    te = jnp.minimum(te, E - 1)

    d0_ref[...] = d0.astype(jnp.int32)
    d1_ref[...] = d1.astype(jnp.int32)
    w0_ref[...] = w0
    w1_ref[...] = w1
    te_ref[...] = te


def _router_call(x_flat, wr):
    outs = pl.pallas_call(
        _router_body,
        out_shape=(
            jax.ShapeDtypeStruct((1, T), jnp.int32),
            jax.ShapeDtypeStruct((1, T), jnp.int32),
            jax.ShapeDtypeStruct((1, T), jnp.float32),
            jax.ShapeDtypeStruct((1, T), jnp.float32),
            jax.ShapeDtypeStruct((1, TE_PAD), jnp.int32),
        ),
    )(x_flat, wr)
    return outs


@functools.cache
def _dispatch_sc_kernel():
    @functools.partial(
        pl.kernel,
        mesh=plsc.VectorSubcoreMesh(core_axis_name="c", subcore_axis_name="s"),
        out_type=jax.ShapeDtypeStruct((P, D), jnp.float32),
        scratch_types=[
            pltpu.VMEM((2, CH_D, D), jnp.float32),
            pltpu.VMEM((NCH_D, CH_D), jnp.int32),
            pltpu.VMEM((NCH_D, CH_D), jnp.int32),
            pltpu.SemaphoreType.DMA((2,)),
            pltpu.SemaphoreType.DMA((2,)),
        ],
    )
    def _dispatch_sc(x_hbm, d0_hbm, d1_hbm, xpad_hbm,
                     xb, i0a, i1a, lsem, ssem):
        wid = lax.axis_index("s") * 2 + lax.axis_index("c")
        pltpu.sync_copy(d0_hbm.at[wid], i0a)
        pltpu.sync_copy(d1_hbm.at[wid], i1a)
        loads = [None] * NCH_D
        scats = [None] * NCH_D

        def start_load(c):
            base = wid * (NCH_D * CH_D) + c * CH_D
            return pltpu.async_copy(x_hbm.at[pl.ds(base, CH_D)],
                                    xb.at[c % 2], lsem.at[c % 2])

        loads[0] = start_load(0)
        for c in range(NCH_D):
            s = c % 2
            if c + 1 < NCH_D:
                if c - 1 >= 0:
                    scats[c - 1][0].wait()
                    scats[c - 1][1].wait()
                loads[c + 1] = start_load(c + 1)
            loads[c].wait()
            a = pltpu.async_copy(xb.at[s], xpad_hbm.at[i0a.at[c]], ssem.at[s])
            b = pltpu.async_copy(xb.at[s], xpad_hbm.at[i1a.at[c]], ssem.at[s])
            scats[c] = (a, b)
        for c in (NCH_D - 2, NCH_D - 1):
            scats[c][0].wait()
            scats[c][1].wait()

    return _dispatch_sc


def _ffn_body(te_ref, x_ref, w1_ref, w3_ref, w2_ref, o_ref):
    x = x_ref[...].astype(jnp.bfloat16)
    w1 = w1_ref[0]
    w3 = w3_ref[0]
    w2 = w2_ref[0]
    h1 = lax.dot_general(x, w1, (((1,), (1,)), ((), ())),
                         preferred_element_type=jnp.float32)
    h3 = lax.dot_general(x, w3, (((1,), (1,)), ((), ())),
                         preferred_element_type=jnp.float32)
    h = (h1 * jax.nn.sigmoid(h1) * h3).astype(jnp.bfloat16)
    y = lax.dot_general(h, w2, (((1,), (1,)), ((), ())),
                        preferred_element_type=jnp.float32)
    o_ref[...] = y


def _ffn_call(x_pad, te, w1, w2, w3):
    grid_spec = pltpu.PrefetchScalarGridSpec(
        num_scalar_prefetch=1,
        grid=(NT,),
        in_specs=[
            pl.BlockSpec((R, D), lambda i, te: (i, 0)),
            pl.BlockSpec((1, H, D), lambda i, te: (te[i], 0, 0)),
            pl.BlockSpec((1, H, D), lambda i, te: (te[i], 0, 0)),
            pl.BlockSpec((1, D, H), lambda i, te: (te[i], 0, 0)),
        ],
        out_specs=pl.BlockSpec((R, D), lambda i, te: (i, 0)),
    )
    return pl.pallas_call(
        _ffn_body,
        grid_spec=grid_spec,
        out_shape=jax.ShapeDtypeStruct((P, D), jnp.float32),
        compiler_params=pltpu.CompilerParams(
            dimension_semantics=("arbitrary",),
        ),
    )(te, x_pad, w1, w3, w2)


@functools.cache
def _combine_sc_kernel():
    @functools.partial(
        pl.kernel,
        mesh=plsc.VectorSubcoreMesh(core_axis_name="c", subcore_axis_name="s"),
        out_type=jax.ShapeDtypeStruct((T, D), jnp.float32),
        scratch_types=[
            pltpu.VMEM((NCH_C, CH_C), jnp.int32),
            pltpu.VMEM((NCH_C, CH_C), jnp.int32),
            pltpu.VMEM((NCH_C, CH_C), jnp.float32),
            pltpu.VMEM((NCH_C, CH_C), jnp.float32),
            pltpu.VMEM((2, CH_C, D), jnp.float32),
            pltpu.VMEM((2, CH_C, D), jnp.float32),
            pltpu.SemaphoreType.DMA((2,)),
            pltpu.SemaphoreType.DMA((2,)),
        ],
    )
    def _combine_sc(yp_hbm, d0_hbm, d1_hbm, w0_hbm, w1_hbm, out_hbm,
                    i0a, i1a, w0a, w1a, b0, b1, gsem, osem):
        wid = lax.axis_index("s") * 2 + lax.axis_index("c")
        pltpu.sync_copy(d0_hbm.at[wid], i0a)
        pltpu.sync_copy(d1_hbm.at[wid], i1a)
        pltpu.sync_copy(w0_hbm.at[wid], w0a)
        pltpu.sync_copy(w1_hbm.at[wid], w1a)

        gath = [None] * NCH_C
        outw = [None] * NCH_C

        def start_gathers(c):
            s = c % 2
            a = pltpu.async_copy(yp_hbm.at[i0a.at[c]], b0.at[s], gsem.at[s])
            b = pltpu.async_copy(yp_hbm.at[i1a.at[c]], b1.at[s], gsem.at[s])
            return (a, b)

        gath[0] = start_gathers(0)
        for c in range(NCH_C):
            s = c % 2
            if c + 1 < NCH_C:
                if c - 1 >= 0:
                    outw[c - 1].wait()
                gath[c + 1] = start_gathers(c + 1)
            gath[c][0].wait()
            gath[c][1].wait()

            w0vec = w0a[c]
            w1vec = w1a[c]
            ws0 = [w0vec[r] for r in range(CH_C)]
            ws1 = [w1vec[r] for r in range(CH_C)]

            def body(cc, _):
                sl = pl.ds(cc * 16, 16)
                for r in range(CH_C):
                    b0[s, r, sl] = ws0[r] * b0[s, r, sl] + ws1[r] * b1[s, r, sl]
                return 0

            lax.fori_loop(0, D // 16, body, 0, unroll=2)
            base = wid * (NCH_C * CH_C) + c * CH_C
            outw[c] = pltpu.async_copy(b0.at[s], out_hbm.at[pl.ds(base, CH_C)],
                                       osem.at[s])
        outw[NCH_C - 2].wait()
        outw[NCH_C - 1].wait()

    return _combine_sc


def kernel(x, Wr, W1, W2, W3):
    b, s, d = x.shape
    x_flat = x.reshape(T, D)
    d0, d1, w0, w1, te = _router_call(x_flat, Wr)

    d0d = d0.reshape(NW, NCH_D, CH_D)
    d1d = d1.reshape(NW, NCH_D, CH_D)
    x_pad = _dispatch_sc_kernel()(x_flat, d0d, d1d)

    y = _ffn_call(x_pad, te.reshape(TE_PAD),
                  W1.astype(jnp.bfloat16),
                  W2.astype(jnp.bfloat16),
                  W3.astype(jnp.bfloat16))

    d0c = d0.reshape(NW, NCH_C, CH_C)
    d1c = d1.reshape(NW, NCH_C, CH_C)
    w0c = w0.reshape(NW, NCH_C, CH_C)
    w1c = w1.reshape(NW, NCH_C, CH_C)
    out = _combine_sc_kernel()(y, d0c, d1c, w0c, w1c)
    return out.reshape(b, s, d)
